# baseline (speedup 1.0000x reference)
import dataclasses

import jax
import jax.numpy as jnp
from jax.experimental import pallas as pl
from jax.experimental.pallas import tpu as pltpu
from jax.experimental.pallas import tpu_sc as plsc

_K = 2048
_N = 2048
_NJ = 256
_BR = 4
_LANES = 16


def _sc_dequant_body(idx_hbm, table_hbm, out_hbm, table_vmem):
    pltpu.sync_copy(table_hbm, table_vmem)

    def body(idx_vmem, out_vmem):
        @pl.loop(0, _BR)
        def _row(r):
            @pl.loop(0, _K, step=_LANES)
            def _col(c):
                ii = idx_vmem[r, pl.ds(c, _LANES)]
                out_vmem[r, pl.ds(c, _LANES)] = plsc.load_gather(
                    table_vmem, [ii])

    pltpu.emit_pipeline(
        body,
        grid=(_N // _BR,),
        in_specs=[pl.BlockSpec((_BR, _K), lambda i: (i, 0))],
        out_specs=[pl.BlockSpec((_BR, _K), lambda i: (i, 0))],
        core_axis_name=("core", "subcore"),
        dimension_semantics=(pltpu.PARALLEL,),
    )(idx_hbm, out_hbm)


def _sc_dequant(weight, weight_table):
    mesh = plsc.VectorSubcoreMesh(
        core_axis_name="core", subcore_axis_name="subcore")
    cp = pltpu.CompilerParams()
    if "needs_layout_passes" in pltpu.CompilerParams.__dataclass_fields__:
        cp = dataclasses.replace(cp, needs_layout_passes=False)
    f = pl.kernel(
        _sc_dequant_body,
        out_type=jax.ShapeDtypeStruct((_N, _K), jnp.float32),
        mesh=mesh,
        scratch_types=[pltpu.VMEM((256,), jnp.float32)],
        compiler_params=cp,
    )
    return f(weight, weight_table.reshape(256))


def _lut(table, idx):
    r = idx.shape[0]
    t_lo = jnp.broadcast_to(table[:, :128], (r, 128))
    t_hi = jnp.broadcast_to(table[:, 128:], (r, 128))
    low = idx & 127
    lo = jnp.take_along_axis(t_lo, low, axis=1, mode="promise_in_bounds")
    hi = jnp.take_along_axis(t_hi, low, axis=1, mode="promise_in_bounds")
    return jnp.where(idx < 128, lo, hi)


def _matmul(x_ref, w_ref, bidx_ref, bt_ref, out_ref):
    wb = w_ref[...].astype(jnp.bfloat16)
    acc = jax.lax.dot_general(
        x_ref[...], wb, (((1,), (1,)), ((), ())),
        preferred_element_type=jnp.float32)
    bidx8 = jnp.broadcast_to(bidx_ref[0], (8, _NJ))
    bvec = _lut(bt_ref[...], bidx8)
    out_ref[...] = acc + bvec[0:1, :]


def kernel(input_, weight, weight_table, bias, bias_table):
    B, M0, K = input_.shape
    M = B * M0
    x = input_.reshape(M, K).astype(jnp.bfloat16)
    w_deq = _sc_dequant(weight, weight_table)
    bt = bias_table.reshape(1, 256)
    J = _N // _NJ
    bidx = bias.reshape(J, 1, _NJ)
    out = pl.pallas_call(
        _matmul,
        grid=(J,),
        in_specs=[
            pl.BlockSpec((M, _K), lambda j: (0, 0)),
            pl.BlockSpec((_NJ, _K), lambda j: (j, 0)),
            pl.BlockSpec((1, 1, _NJ), lambda j: (j, 0, 0)),
            pl.BlockSpec((1, 256), lambda j: (0, 0)),
        ],
        out_specs=pl.BlockSpec((M, _NJ), lambda j: (0, j)),
        out_shape=jax.ShapeDtypeStruct((M, _N), jnp.float32),
    )(x, w_deq, bidx, bt)
    return out.reshape(B, M0, _N)


# speedup vs baseline: 2.4211x; 2.4211x over previous
import jax
import jax.numpy as jnp
from jax.experimental import pallas as pl
from jax.experimental.pallas import tpu as pltpu

_K = 2048
_N = 2048
_NJ = 256
_NI = 2048


def _lut(table, idx):
    r = idx.shape[0]
    t_lo = jnp.broadcast_to(table[:, :128], (r, 128))
    t_hi = jnp.broadcast_to(table[:, 128:], (r, 128))
    low = idx & 127
    lo = jnp.take_along_axis(t_lo, low, axis=1, mode="promise_in_bounds")
    hi = jnp.take_along_axis(t_hi, low, axis=1, mode="promise_in_bounds")
    return jnp.where(idx < 128, lo, hi)


def _fused(x_ref, idx_ref, wt_ref, bidx_ref, bt_ref, out_ref, xb_ref):
    j = pl.program_id(1)

    @pl.when(j == 0)
    def _cast():
        xb_ref[...] = x_ref[...].astype(jnp.bfloat16)

    idx = idx_ref[...]
    w = _lut(wt_ref[...], idx)
    wb = w.astype(jnp.bfloat16)
    acc = jax.lax.dot_general(
        xb_ref[...], wb, (((1,), (1,)), ((), ())),
        preferred_element_type=jnp.float32)
    bidx8 = jnp.broadcast_to(bidx_ref[0], (8, _NJ))
    bvec = _lut(bt_ref[...], bidx8)
    out_ref[...] = acc + bvec[0:1, :]


def kernel(input_, weight, weight_table, bias, bias_table):
    B, M0, K = input_.shape
    M = B * M0
    x = input_.reshape(M, K)
    wt = weight_table.reshape(1, 256)
    bt = bias_table.reshape(1, 256)
    J = _N // _NJ
    I = M // _NI
    bidx = bias.reshape(J, 1, _NJ)
    out = pl.pallas_call(
        _fused,
        grid=(I, J),
        in_specs=[
            pl.BlockSpec((_NI, _K), lambda i, j: (i, 0)),
            pl.BlockSpec((_NJ, _K), lambda i, j: (j, 0)),
            pl.BlockSpec((1, 256), lambda i, j: (0, 0)),
            pl.BlockSpec((1, 1, _NJ), lambda i, j: (j, 0, 0)),
            pl.BlockSpec((1, 256), lambda i, j: (0, 0)),
        ],
        out_specs=pl.BlockSpec((_NI, _NJ), lambda i, j: (i, j)),
        out_shape=jax.ShapeDtypeStruct((M, _N), jnp.float32),
        scratch_shapes=[pltpu.VMEM((_NI, _K), jnp.bfloat16)],
    )(x, weight, wt, bidx, bt)
    return out.reshape(B, M0, _N)
